# baseline (speedup 1.0000x reference)
import jax
import jax.numpy as jnp
from jax import lax
from jax.experimental import pallas as pl
from jax.experimental.pallas import tpu as pltpu
from jax.experimental.pallas import tpu_sc as plsc

NC = 2
NS = 16
L = 16
NW = NC * NS

D = 64
STEP = 128
NB = 5
P = 3
_ENABLE_ZERO = True


def _gather_body(W_hbm, x_hbm, out_hbm, *scratch):
    idx_v = scratch[0]
    bufs = scratch[1:1 + NB]
    in_sems = scratch[1 + NB:1 + 2 * NB]
    out_sems = scratch[1 + 2 * NB:1 + 3 * NB]

    bpw = x_hbm.shape[0] // NW
    nstep = bpw // STEP

    wid = lax.axis_index("s") * NC + lax.axis_index("c")
    base = wid * bpw
    pltpu.sync_copy(x_hbm.at[pl.ds(base, bpw)], idx_v)

    def start_gather(step, b):
        pltpu.async_copy(
            W_hbm.at[idx_v.at[pl.ds(step * STEP, STEP)]], bufs[b], in_sems[b])

    def wait_gather(b):
        pltpu.make_async_copy(W_hbm.at[pl.ds(0, STEP)], bufs[b], in_sems[b]).wait()

    def start_out(step, b):
        pltpu.async_copy(
            bufs[b], out_hbm.at[pl.ds(base + step * STEP, STEP)], out_sems[b])

    def wait_out(b):
        pltpu.make_async_copy(bufs[b], out_hbm.at[pl.ds(0, STEP)], out_sems[b]).wait()

    rows = [lax.broadcasted_iota(jnp.int32, (L,), 0) + k * L
            for k in range(STEP // L)]
    zeros_f = jnp.zeros((L,), jnp.float32)

    def zero_pad_rows(g, b):
        j0 = g * STEP
        acc = jnp.zeros((L,), jnp.int32)
        for k in range(STEP // L):
            vec = idx_v[pl.ds(j0 + k * L, L)]
            acc = acc + (vec == 0).astype(jnp.int32)
        cnt = jnp.sum(acc)

        @pl.when(cnt != 0)
        def _slow():
            for k in range(STEP // L):
                vec = idx_v[pl.ds(j0 + k * L, L)]
                mk = vec == 0
                ck = jnp.sum(mk.astype(jnp.int32))

                @pl.when(ck != 0)
                def _zero_group():
                    def col_body(c, carry):
                        cols = jnp.zeros((L,), jnp.int32) + c
                        plsc.store_scatter(bufs[b], [rows[k], cols], zeros_f,
                                           mask=mk)
                        return carry
                    lax.fori_loop(0, D, col_body, 0)

    for j in range(P):
        start_gather(j, j)

    def outer(o, carry):
        for k in range(NB):
            g = o * NB + k
            nxt = g + P
            bn = (k + P) % NB

            @pl.when(nxt < nstep)
            def _issue():
                @pl.when(nxt >= NB)
                def _reuse_wait():
                    wait_out(bn)
                start_gather(nxt, bn)

            wait_gather(k)
            if _ENABLE_ZERO:
                zero_pad_rows(g, k)
            start_out(g, k)
        return carry

    lax.fori_loop(0, nstep // NB, outer, 0)

    for b in range(NB):
        wait_out(b)


def kernel(x, W):
    n_rows, n_cols = x.shape
    B = n_rows * n_cols
    xf = x.reshape(B).astype(jnp.int32)

    mesh = plsc.VectorSubcoreMesh(
        core_axis_name="c", subcore_axis_name="s",
        num_cores=NC, num_subcores=NS)
    bpw = B // NW
    scratch = (
        [pltpu.VMEM((bpw,), jnp.int32)]
        + [pltpu.VMEM((STEP, D), jnp.float32) for _ in range(NB)]
        + [pltpu.SemaphoreType.DMA] * (2 * NB)
    )
    gather = pl.kernel(
        _gather_body,
        out_type=jax.ShapeDtypeStruct((B, D), jnp.float32),
        mesh=mesh,
        scratch_types=scratch,
        compiler_params=pltpu.CompilerParams(
            use_tc_tiling_on_sc=False, needs_layout_passes=False),
    )
    out = gather(W, xf)
    return out.reshape(n_rows, n_cols, D)
